# baseline (speedup 1.0000x reference)
import functools

import jax
import jax.numpy as jnp
from jax import lax
from jax.experimental import pallas as pl
from jax.experimental.pallas import tpu as pltpu
from jax.experimental.pallas import tpu_sc as plsc

N = 10000
E = 320000
F = 128

NC = 2
NS = 16
NW = NC * NS
EPT = E // NW
CH = 80
NCHUNK = EPT // CH
RPT = 624
TAIL = N - NS * RPT

ROW_BLK = 1000


def _tc1a_body(x_ref, w1_ref, apad_ref, s2_ref):
    v = jnp.dot(w1_ref[...], apad_ref[...], preferred_element_type=jnp.float32)
    s2_ref[...] = jnp.dot(x_ref[...], v, preferred_element_type=jnp.float32)


def _tc1b_body(x_ref, w1_ref, h1_ref):
    h1_ref[...] = jnp.dot(x_ref[...], w1_ref[...],
                          preferred_element_type=jnp.float32)


def _tc2_body(a0_ref, a1_ref, w2_ref, h2_ref):
    h = jnp.maximum(a0_ref[...] + a1_ref[...], 0.0)
    h2_ref[...] = jnp.dot(h, w2_ref[...], preferred_element_type=jnp.float32)


def _tc3_body(a0_ref, a1_ref, x_ref, lnw_ref, lnb_ref, o_ref):
    h = jnp.maximum(a0_ref[...] + a1_ref[...], 0.0) + x_ref[...]
    m = jnp.mean(h, axis=-1, keepdims=True)
    cen = h - m
    var = jnp.mean(cen * cen, axis=-1, keepdims=True)
    o_ref[...] = cen * lax.rsqrt(var + 1e-5) * lnw_ref[...] + lnb_ref[...]


def _row_spec():
    return pl.BlockSpec((ROW_BLK, F), lambda i: (i, 0))


def _full_spec():
    return pl.BlockSpec((F, F), lambda i: (0, 0))


def _vec_spec():
    return pl.BlockSpec((1, F), lambda i: (0, 0))


NSET = 4


def _scatter_pipeline(h_hbm, acc_sh, packed_hbm, w_hbm, ebase, sets):
    def issue_p(kk, S):
        pltpu.async_copy(packed_hbm.at[pl.ds(ebase + kk * CH, CH)], S[1], S[5])
        pltpu.async_copy(w_hbm.at[pl.ds(ebase + kk * CH, CH)], S[2], S[5])

    def wait_p(S):
        pltpu.make_async_copy(packed_hbm.at[pl.ds(ebase, CH)], S[1], S[5]).wait()
        pltpu.make_async_copy(w_hbm.at[pl.ds(ebase, CH)], S[2], S[5]).wait()

    def unpack(S):
        for i in range(CH // 16):
            sl = pl.ds(i * 16, 16)
            p = S[1][sl]
            S[3][sl] = jnp.bitwise_and(p, 0xFFFF)
            S[4][sl] = lax.shift_right_logical(p, 16)

    def issue_g(S):
        pltpu.async_copy(h_hbm.at[S[4]], S[0], S[6])

    def wait_g(S):
        pltpu.make_async_copy(h_hbm.at[S[4]], S[0], S[6]).wait()

    def issue_a(S):
        pltpu.async_copy(S[0], acc_sh.at[S[3]], S[7], add=True)

    def wait_a(S):
        pltpu.make_async_copy(S[0], acc_sh.at[S[3]], S[7]).wait()

    def scale(S):
        rows = S[0]
        for i in range(CH // 16):
            wv = S[2][pl.ds(i * 16, 16)]
            for l in range(16):
                wb = jnp.full((16,), wv[l], dtype=jnp.float32)
                e = i * 16 + l
                for j in range(F // 16):
                    sl = pl.ds(j * 16, 16)
                    rows[e, sl] = rows[e, sl] * wb

    def step(k, s0, s1, s2, wait_prev_a):
        @pl.when(k + 1 < NCHUNK)
        def _():
            if wait_prev_a:
                wait_a(s1)
            wait_p(s1)
            unpack(s1)
            issue_g(s1)

        @pl.when(k + 2 < NCHUNK)
        def _():
            issue_p(k + 2, s2)

        @pl.when(k < NCHUNK)
        def _():
            wait_g(s0)
            scale(s0)
            issue_a(s0)

    def rot(k):
        return (sets[k % NSET], sets[(k + 1) % NSET], sets[(k + 2) % NSET])

    issue_p(0, sets[0])
    issue_p(1, sets[1])
    wait_p(sets[0])
    unpack(sets[0])
    issue_g(sets[0])
    for k in range(NSET - 1):
        step(k, *rot(k), False)

    def quad(t, _):
        k = NSET - 1 + NSET * t
        for d in range(NSET):
            step(k + d, *rot(NSET - 1 + d), True)
        return 0

    nquad = (NCHUNK - (NSET - 1) + NSET - 1) // NSET
    lax.fori_loop(0, nquad, quad, 0)
    for k in range(NCHUNK - NSET, NCHUNK):
        wait_a(sets[k % NSET])


def _zero_acc(zeros, acc_sh, s):
    pltpu.sync_copy(zeros.at[pl.ds(s * RPT, RPT)], acc_sh.at[pl.ds(s * RPT, RPT)])

    @pl.when(s == NS - 1)
    def _():
        pltpu.sync_copy(zeros.at[pl.ds(NS * RPT, TAIL)],
                        acc_sh.at[pl.ds(NS * RPT, TAIL)])


def _drain_acc(acc_sh, acc_out, c, s):
    pltpu.sync_copy(acc_sh.at[pl.ds(s * RPT, RPT)],
                    acc_out.at[c, pl.ds(s * RPT, RPT)])

    @pl.when(s == NS - 1)
    def _():
        pltpu.sync_copy(acc_sh.at[pl.ds(NS * RPT, TAIL)],
                        acc_out.at[c, pl.ds(NS * RPT, TAIL)])


def _scw_body(packed, adj, st, sb, w_out,
              packed_v, adj_v, st_v, sb_v, w_v,
              sem0, sem1, sem2, sem3):
    c = lax.axis_index("c")
    s = lax.axis_index("s")
    wid = s * NC + c
    ebase = wid * EPT
    d0 = pltpu.async_copy(st, st_v, sem0)
    d1 = pltpu.async_copy(sb, sb_v, sem1)
    d2 = pltpu.async_copy(packed.at[pl.ds(ebase, EPT)], packed_v, sem2)
    d3 = pltpu.async_copy(adj.at[pl.ds(ebase, EPT)], adj_v, sem3)
    d0.wait()
    d1.wait()
    d2.wait()
    d3.wait()

    def wbody(i, _):
        sl = pl.ds(i * 16, 16)
        p = packed_v[sl]
        sT = plsc.load_gather(st_v, [jnp.bitwise_and(p, 0xFFFF)])
        sB = plsc.load_gather(sb_v, [lax.shift_right_logical(p, 16)])
        sc = sT + sB
        sc = jnp.maximum(sc, 0.2 * sc)
        w_v[sl] = adj_v[sl] / (1.0 + jnp.exp(-sc))
        return 0

    lax.fori_loop(0, EPT // 16, wbody, 0)
    pltpu.sync_copy(w_v, w_out.at[pl.ds(ebase, EPT)])


def _scat_body(h, packed, w_in, zeros, acc_out, *rest):
    refs, sems = rest[:5 * NSET + 1], rest[5 * NSET + 1:]
    acc_sh = refs[5 * NSET]
    sets = tuple(
        (refs[i], refs[NSET + i], refs[2 * NSET + i], refs[3 * NSET + i],
         refs[4 * NSET + i], sems[i], sems[NSET + i], sems[2 * NSET + i])
        for i in range(NSET))
    c = lax.axis_index("c")
    s = lax.axis_index("s")
    wid = s * NC + c
    ebase = wid * EPT
    _zero_acc(zeros, acc_sh, s)
    plsc.subcore_barrier()
    _scatter_pipeline(h, acc_sh, packed, w_in, ebase, sets)
    plsc.subcore_barrier()
    _drain_acc(acc_sh, acc_out, c, s)


@functools.cache
def _build():
    f32 = jnp.float32
    mesh = plsc.VectorSubcoreMesh(core_axis_name="c", subcore_axis_name="s",
                                  num_cores=NC, num_subcores=NS)

    tc1a = pl.pallas_call(
        _tc1a_body,
        grid=(N // ROW_BLK,),
        in_specs=[_row_spec(), _full_spec(), _full_spec()],
        out_specs=_row_spec(),
        out_shape=jax.ShapeDtypeStruct((N, F), f32),
    )

    tc1b = pl.pallas_call(
        _tc1b_body,
        grid=(N // ROW_BLK,),
        in_specs=[_row_spec(), _full_spec()],
        out_specs=_row_spec(),
        out_shape=jax.ShapeDtypeStruct((N, F), f32),
    )

    sc_params = pltpu.CompilerParams(needs_layout_passes=False)

    scw = pl.kernel(
        _scw_body,
        out_type=jax.ShapeDtypeStruct((E,), f32),
        mesh=mesh,
        compiler_params=sc_params,
        scratch_types=(
            [pltpu.VMEM((EPT,), jnp.int32),
             pltpu.VMEM((EPT,), f32),
             pltpu.VMEM((N,), f32),
             pltpu.VMEM((N,), f32),
             pltpu.VMEM((EPT,), f32)]
            + [pltpu.SemaphoreType.DMA] * 4
        ),
    )

    tc2 = pl.pallas_call(
        _tc2_body,
        grid=(N // ROW_BLK,),
        in_specs=[_row_spec(), _row_spec(), _full_spec()],
        out_specs=_row_spec(),
        out_shape=jax.ShapeDtypeStruct((N, F), f32),
    )

    scat = pl.kernel(
        _scat_body,
        out_type=jax.ShapeDtypeStruct((NC, N, F), f32),
        mesh=mesh,
        compiler_params=sc_params,
        scratch_types=(
            [pltpu.VMEM((CH, F), f32)] * NSET
            + [pltpu.VMEM((CH,), jnp.int32)] * NSET
            + [pltpu.VMEM((CH,), f32)] * NSET
            + [pltpu.VMEM((CH,), jnp.int32)] * NSET
            + [pltpu.VMEM((CH,), jnp.int32)] * NSET
            + [pltpu.VMEM_SHARED((N, F), f32)]
            + [pltpu.SemaphoreType.DMA] * (3 * NSET)
        ),
    )

    tc3 = pl.pallas_call(
        _tc3_body,
        grid=(N // ROW_BLK,),
        in_specs=[_row_spec(), _row_spec(), _row_spec(), _vec_spec(), _vec_spec()],
        out_specs=_row_spec(),
        out_shape=jax.ShapeDtypeStruct((N, F), f32),
    )

    return tc1a, tc1b, scw, scat, tc2, tc3


def kernel(x, edge_index, adj_vals, W1, a1, W2, ln_w, ln_b):
    tc1a, tc1b, scw, scat, tc2, tc3 = _build()
    f32 = jnp.float32
    row = edge_index[0]
    col = edge_index[1]
    packed = jnp.bitwise_or(row, jnp.left_shift(col, 16))
    a_flat = a1[:, 0]
    a_pad = jnp.zeros((F, F), f32).at[:, 0].set(a_flat[:F]).at[:, 1].set(a_flat[F:])
    zeros = jnp.zeros((N, F), f32)

    s2 = tc1a(x, W1, a_pad)
    st = s2[:, 0]
    sb = s2[:, 1]
    w = scw(packed, adj_vals, st, sb)
    h1 = tc1b(x, W1)
    acc = scat(h1, packed, w, zeros)
    h2 = tc2(acc[0], acc[1], W2)
    acc2 = scat(h2, packed, w, zeros)
    out = tc3(acc2[0], acc2[1], x, ln_w.reshape(1, F), ln_b.reshape(1, F))
    return out


# speedup vs baseline: 1.0102x; 1.0102x over previous
import functools

import jax
import jax.numpy as jnp
from jax import lax
from jax.experimental import pallas as pl
from jax.experimental.pallas import tpu as pltpu
from jax.experimental.pallas import tpu_sc as plsc

N = 10000
E = 320000
F = 128

NC = 2
NS = 16
NW = NC * NS
EPT = E // NW
CH = 80
NCHUNK = EPT // CH
RPT = 624
TAIL = N - NS * RPT

ROW_BLK = 1000


def _tc1a_body(x_ref, w1_ref, apad_ref, s2_ref):
    v = jnp.dot(w1_ref[...], apad_ref[...], preferred_element_type=jnp.float32)
    s2_ref[...] = jnp.dot(x_ref[...], v, preferred_element_type=jnp.float32)


def _tc1b_body(x_ref, w1_ref, h1_ref):
    h1_ref[...] = jnp.dot(x_ref[...], w1_ref[...],
                          preferred_element_type=jnp.float32)


def _tc2_body(a0_ref, a1_ref, w2_ref, h2_ref):
    h = jnp.maximum(a0_ref[...] + a1_ref[...], 0.0)
    h2_ref[...] = jnp.dot(h, w2_ref[...], preferred_element_type=jnp.float32)


def _tc3_body(a0_ref, a1_ref, x_ref, lnw_ref, lnb_ref, o_ref):
    h = jnp.maximum(a0_ref[...] + a1_ref[...], 0.0) + x_ref[...]
    m = jnp.mean(h, axis=-1, keepdims=True)
    cen = h - m
    var = jnp.mean(cen * cen, axis=-1, keepdims=True)
    o_ref[...] = cen * lax.rsqrt(var + 1e-5) * lnw_ref[...] + lnb_ref[...]


def _row_spec():
    return pl.BlockSpec((ROW_BLK, F), lambda i: (i, 0))


def _full_spec():
    return pl.BlockSpec((F, F), lambda i: (0, 0))


def _vec_spec():
    return pl.BlockSpec((1, F), lambda i: (0, 0))


NSET = 3


def _scatter_pipeline(h_hbm, acc_sh, packed_hbm, ebase, w_v, sets):
    def issue_p(kk, S):
        pltpu.async_copy(packed_hbm.at[pl.ds(ebase + kk * CH, CH)], S[1], S[4])

    def wait_p(S):
        pltpu.make_async_copy(packed_hbm.at[pl.ds(ebase, CH)], S[1], S[4]).wait()

    def unpack(S):
        for i in range(CH // 16):
            sl = pl.ds(i * 16, 16)
            p = S[1][sl]
            S[2][sl] = jnp.bitwise_and(p, 0xFFFF)
            S[3][sl] = lax.shift_right_logical(p, 16)

    def issue_g(S):
        pltpu.async_copy(h_hbm.at[S[3]], S[0], S[5])

    def wait_g(S):
        pltpu.make_async_copy(h_hbm.at[S[3]], S[0], S[5]).wait()

    def issue_a(S):
        pltpu.async_copy(S[0], acc_sh.at[S[2]], S[6], add=True)

    def wait_a(S):
        pltpu.make_async_copy(S[0], acc_sh.at[S[2]], S[6]).wait()

    def scale(kk, S):
        rows = S[0]
        for i in range(CH // 16):
            wv = w_v[pl.ds(kk * CH + i * 16, 16)]
            for l in range(16):
                wb = jnp.full((16,), wv[l], dtype=jnp.float32)
                e = i * 16 + l
                for j in range(F // 16):
                    sl = pl.ds(j * 16, 16)
                    rows[e, sl] = rows[e, sl] * wb

    def step(k, s0, s1, s2, wait_prev_a):
        @pl.when(k + 1 < NCHUNK)
        def _():
            if wait_prev_a:
                wait_a(s1)
            wait_p(s1)
            unpack(s1)
            issue_g(s1)

        @pl.when(k + 2 < NCHUNK)
        def _():
            issue_p(k + 2, s2)

        @pl.when(k < NCHUNK)
        def _():
            wait_g(s0)
            scale(k, s0)
            issue_a(s0)

    def rot(k):
        return (sets[k % NSET], sets[(k + 1) % NSET], sets[(k + 2) % NSET])

    issue_p(0, sets[0])
    issue_p(1, sets[1])
    wait_p(sets[0])
    unpack(sets[0])
    issue_g(sets[0])
    for k in range(NSET - 1):
        step(k, *rot(k), False)

    def quad(t, _):
        k = NSET - 1 + NSET * t
        for d in range(NSET):
            step(k + d, *rot(NSET - 1 + d), True)
        return 0

    nquad = (NCHUNK - (NSET - 1) + NSET - 1) // NSET
    lax.fori_loop(0, nquad, quad, 0)
    for k in range(NCHUNK - NSET, NCHUNK):
        wait_a(sets[k % NSET])


def _zero_acc(zeros, acc_sh, s):
    pltpu.sync_copy(zeros.at[pl.ds(s * RPT, RPT)], acc_sh.at[pl.ds(s * RPT, RPT)])

    @pl.when(s == NS - 1)
    def _():
        pltpu.sync_copy(zeros.at[pl.ds(NS * RPT, TAIL)],
                        acc_sh.at[pl.ds(NS * RPT, TAIL)])


def _drain_acc(acc_sh, acc_out, c, s):
    pltpu.sync_copy(acc_sh.at[pl.ds(s * RPT, RPT)],
                    acc_out.at[c, pl.ds(s * RPT, RPT)])

    @pl.when(s == NS - 1)
    def _():
        pltpu.sync_copy(acc_sh.at[pl.ds(NS * RPT, TAIL)],
                        acc_out.at[c, pl.ds(NS * RPT, TAIL)])


def _scw_body(packed, adj, st, sb, w_out,
              packed_v, adj_v, st_v, sb_v, w_v,
              sem0, sem1, sem2, sem3):
    c = lax.axis_index("c")
    s = lax.axis_index("s")
    wid = s * NC + c
    ebase = wid * EPT
    d0 = pltpu.async_copy(st, st_v, sem0)
    d1 = pltpu.async_copy(sb, sb_v, sem1)
    d2 = pltpu.async_copy(packed.at[pl.ds(ebase, EPT)], packed_v, sem2)
    d3 = pltpu.async_copy(adj.at[pl.ds(ebase, EPT)], adj_v, sem3)
    d0.wait()
    d1.wait()
    d2.wait()
    d3.wait()

    def wbody(i, _):
        sl = pl.ds(i * 16, 16)
        p = packed_v[sl]
        sT = plsc.load_gather(st_v, [jnp.bitwise_and(p, 0xFFFF)])
        sB = plsc.load_gather(sb_v, [lax.shift_right_logical(p, 16)])
        sc = sT + sB
        sc = jnp.maximum(sc, 0.2 * sc)
        w_v[sl] = adj_v[sl] / (1.0 + jnp.exp(-sc))
        return 0

    lax.fori_loop(0, EPT // 16, wbody, 0)
    pltpu.sync_copy(w_v, w_out.at[pl.ds(ebase, EPT)])


def _scat_body(h, packed, w_in, zeros, acc_out, *rest):
    w_v = rest[0]
    refs, sems = rest[1:4 * NSET + 2], rest[4 * NSET + 2:]
    acc_sh = refs[4 * NSET]
    sets = tuple(
        (refs[i], refs[NSET + i], refs[2 * NSET + i], refs[3 * NSET + i],
         sems[i], sems[NSET + i], sems[2 * NSET + i])
        for i in range(NSET))
    c = lax.axis_index("c")
    s = lax.axis_index("s")
    wid = s * NC + c
    ebase = wid * EPT
    pltpu.sync_copy(w_in.at[pl.ds(ebase, EPT)], w_v)
    _zero_acc(zeros, acc_sh, s)
    plsc.subcore_barrier()
    _scatter_pipeline(h, acc_sh, packed, ebase, w_v, sets)
    plsc.subcore_barrier()
    _drain_acc(acc_sh, acc_out, c, s)


@functools.cache
def _build():
    f32 = jnp.float32
    mesh = plsc.VectorSubcoreMesh(core_axis_name="c", subcore_axis_name="s",
                                  num_cores=NC, num_subcores=NS)

    tc1a = pl.pallas_call(
        _tc1a_body,
        grid=(N // ROW_BLK,),
        in_specs=[_row_spec(), _full_spec(), _full_spec()],
        out_specs=_row_spec(),
        out_shape=jax.ShapeDtypeStruct((N, F), f32),
    )

    tc1b = pl.pallas_call(
        _tc1b_body,
        grid=(N // ROW_BLK,),
        in_specs=[_row_spec(), _full_spec()],
        out_specs=_row_spec(),
        out_shape=jax.ShapeDtypeStruct((N, F), f32),
    )

    sc_params = pltpu.CompilerParams(needs_layout_passes=False)

    scw = pl.kernel(
        _scw_body,
        out_type=jax.ShapeDtypeStruct((E,), f32),
        mesh=mesh,
        compiler_params=sc_params,
        scratch_types=(
            [pltpu.VMEM((EPT,), jnp.int32),
             pltpu.VMEM((EPT,), f32),
             pltpu.VMEM((N,), f32),
             pltpu.VMEM((N,), f32),
             pltpu.VMEM((EPT,), f32)]
            + [pltpu.SemaphoreType.DMA] * 4
        ),
    )

    tc2 = pl.pallas_call(
        _tc2_body,
        grid=(N // ROW_BLK,),
        in_specs=[_row_spec(), _row_spec(), _full_spec()],
        out_specs=_row_spec(),
        out_shape=jax.ShapeDtypeStruct((N, F), f32),
    )

    scat = pl.kernel(
        _scat_body,
        out_type=jax.ShapeDtypeStruct((NC, N, F), f32),
        mesh=mesh,
        compiler_params=sc_params,
        scratch_types=(
            [pltpu.VMEM((EPT,), f32)]
            + [pltpu.VMEM((CH, F), f32)] * NSET
            + [pltpu.VMEM((CH,), jnp.int32)] * NSET
            + [pltpu.VMEM((CH,), jnp.int32)] * NSET
            + [pltpu.VMEM((CH,), jnp.int32)] * NSET
            + [pltpu.VMEM_SHARED((N, F), f32)]
            + [pltpu.SemaphoreType.DMA] * (3 * NSET)
        ),
    )

    tc3 = pl.pallas_call(
        _tc3_body,
        grid=(N // ROW_BLK,),
        in_specs=[_row_spec(), _row_spec(), _row_spec(), _vec_spec(), _vec_spec()],
        out_specs=_row_spec(),
        out_shape=jax.ShapeDtypeStruct((N, F), f32),
    )

    return tc1a, tc1b, scw, scat, tc2, tc3


def kernel(x, edge_index, adj_vals, W1, a1, W2, ln_w, ln_b):
    tc1a, tc1b, scw, scat, tc2, tc3 = _build()
    f32 = jnp.float32
    row = edge_index[0]
    col = edge_index[1]
    packed = jnp.bitwise_or(row, jnp.left_shift(col, 16))
    a_flat = a1[:, 0]
    a_pad = jnp.zeros((F, F), f32).at[:, 0].set(a_flat[:F]).at[:, 1].set(a_flat[F:])
    zeros = jnp.zeros((N, F), f32)

    s2 = tc1a(x, W1, a_pad)
    st = s2[:, 0]
    sb = s2[:, 1]
    w = scw(packed, adj_vals, st, sb)
    h1 = tc1b(x, W1)
    acc = scat(h1, packed, w, zeros)
    h2 = tc2(acc[0], acc[1], W2)
    acc2 = scat(h2, packed, w, zeros)
    out = tc3(acc2[0], acc2[1], x, ln_w.reshape(1, F), ln_b.reshape(1, F))
    return out


# speedup vs baseline: 1.0124x; 1.0022x over previous
import functools

import jax
import jax.numpy as jnp
from jax import lax
from jax.experimental import pallas as pl
from jax.experimental.pallas import tpu as pltpu
from jax.experimental.pallas import tpu_sc as plsc

N = 10000
E = 320000
F = 128

NC = 2
NS = 16
NW = NC * NS
EPT = E // NW
CH = 80
NCHUNK = EPT // CH
RPT = 624
TAIL = N - NS * RPT

ROW_BLK = 1000


def _tc1a_body(x_ref, w1_ref, apad_ref, s2_ref):
    v = jnp.dot(w1_ref[...], apad_ref[...], preferred_element_type=jnp.float32)
    s2_ref[...] = jnp.dot(x_ref[...], v, preferred_element_type=jnp.float32)


def _tc1b_body(x_ref, w1_ref, h1_ref):
    h1_ref[...] = jnp.dot(x_ref[...], w1_ref[...],
                          preferred_element_type=jnp.float32)


def _tc2_body(a0_ref, a1_ref, w2_ref, h2_ref):
    h = jnp.maximum(a0_ref[...] + a1_ref[...], 0.0)
    h2_ref[...] = jnp.dot(h, w2_ref[...], preferred_element_type=jnp.float32)


def _tc3_body(a0_ref, a1_ref, x_ref, lnw_ref, lnb_ref, o_ref):
    h = jnp.maximum(a0_ref[...] + a1_ref[...], 0.0) + x_ref[...]
    m = jnp.mean(h, axis=-1, keepdims=True)
    cen = h - m
    var = jnp.mean(cen * cen, axis=-1, keepdims=True)
    o_ref[...] = cen * lax.rsqrt(var + 1e-5) * lnw_ref[...] + lnb_ref[...]


def _row_spec():
    return pl.BlockSpec((ROW_BLK, F), lambda i: (i, 0))


def _full_spec():
    return pl.BlockSpec((F, F), lambda i: (0, 0))


def _vec_spec():
    return pl.BlockSpec((1, F), lambda i: (0, 0))


NSET = 3


def _scatter_pipeline(h_hbm, acc_sh, packed_hbm, ebase, w_v, sets):
    def issue_p(kk, S):
        pltpu.async_copy(packed_hbm.at[pl.ds(ebase + kk * CH, CH)], S[1], S[4])

    def wait_p(S):
        pltpu.make_async_copy(packed_hbm.at[pl.ds(ebase, CH)], S[1], S[4]).wait()

    def unpack(S):
        for i in range(CH // 16):
            sl = pl.ds(i * 16, 16)
            p = S[1][sl]
            S[2][sl] = jnp.bitwise_and(p, 0xFFFF)
            S[3][sl] = lax.shift_right_logical(p, 16)

    def issue_g(S):
        pltpu.async_copy(h_hbm.at[S[3]], S[0], S[5])

    def wait_g(S):
        pltpu.make_async_copy(h_hbm.at[S[3]], S[0], S[5]).wait()

    def issue_a(S):
        pltpu.async_copy(S[0], acc_sh.at[S[2]], S[6], add=True)

    def wait_a(S):
        pltpu.make_async_copy(S[0], acc_sh.at[S[2]], S[6]).wait()

    def scale(kk, S):
        rows = S[0]
        for i in range(CH // 16):
            wv = w_v[pl.ds(kk * CH + i * 16, 16)]
            for l in range(16):
                wb = jnp.full((16,), wv[l], dtype=jnp.float32)
                e = i * 16 + l
                for j in range(F // 16):
                    sl = pl.ds(j * 16, 16)
                    rows[e, sl] = rows[e, sl] * wb

    def step(k, s0, s1, s2, wait_prev_a):
        @pl.when(k + 1 < NCHUNK)
        def _():
            if wait_prev_a:
                wait_a(s1)
            wait_p(s1)
            unpack(s1)
            issue_g(s1)

        @pl.when(k + 2 < NCHUNK)
        def _():
            issue_p(k + 2, s2)

        wait_g(s0)
        scale(k, s0)
        issue_a(s0)

    def rot(k):
        return (sets[k % NSET], sets[(k + 1) % NSET], sets[(k + 2) % NSET])

    issue_p(0, sets[0])
    issue_p(1, sets[1])
    wait_p(sets[0])
    unpack(sets[0])
    issue_g(sets[0])
    for k in range(NSET - 1):
        step(k, *rot(k), False)

    def quad(t, _):
        k = NSET - 1 + NSET * t
        for d in range(NSET):
            step(k + d, *rot(NSET - 1 + d), True)
        return 0

    assert (NCHUNK - (NSET - 1)) % NSET == 0
    lax.fori_loop(0, (NCHUNK - (NSET - 1)) // NSET, quad, 0)
    for k in range(NCHUNK - NSET, NCHUNK):
        wait_a(sets[k % NSET])


def _zero_acc(zeros, acc_sh, s):
    pltpu.sync_copy(zeros.at[pl.ds(s * RPT, RPT)], acc_sh.at[pl.ds(s * RPT, RPT)])

    @pl.when(s == NS - 1)
    def _():
        pltpu.sync_copy(zeros.at[pl.ds(NS * RPT, TAIL)],
                        acc_sh.at[pl.ds(NS * RPT, TAIL)])


def _drain_acc(acc_sh, acc_out, c, s):
    pltpu.sync_copy(acc_sh.at[pl.ds(s * RPT, RPT)],
                    acc_out.at[c, pl.ds(s * RPT, RPT)])

    @pl.when(s == NS - 1)
    def _():
        pltpu.sync_copy(acc_sh.at[pl.ds(NS * RPT, TAIL)],
                        acc_out.at[c, pl.ds(NS * RPT, TAIL)])


def _scw_body(packed, adj, st, sb, w_out,
              packed_v, adj_v, st_v, sb_v, w_v,
              sem0, sem1, sem2, sem3):
    c = lax.axis_index("c")
    s = lax.axis_index("s")
    wid = s * NC + c
    ebase = wid * EPT
    d0 = pltpu.async_copy(st, st_v, sem0)
    d1 = pltpu.async_copy(sb, sb_v, sem1)
    d2 = pltpu.async_copy(packed.at[pl.ds(ebase, EPT)], packed_v, sem2)
    d3 = pltpu.async_copy(adj.at[pl.ds(ebase, EPT)], adj_v, sem3)
    d0.wait()
    d1.wait()
    d2.wait()
    d3.wait()

    def wbody(i, _):
        sl = pl.ds(i * 16, 16)
        p = packed_v[sl]
        sT = plsc.load_gather(st_v, [jnp.bitwise_and(p, 0xFFFF)])
        sB = plsc.load_gather(sb_v, [lax.shift_right_logical(p, 16)])
        sc = sT + sB
        sc = jnp.maximum(sc, 0.2 * sc)
        w_v[sl] = adj_v[sl] / (1.0 + jnp.exp(-sc))
        return 0

    lax.fori_loop(0, EPT // 16, wbody, 0)
    pltpu.sync_copy(w_v, w_out.at[pl.ds(ebase, EPT)])


def _scat_body(h, packed, w_in, zeros, acc_out, *rest):
    w_v = rest[0]
    refs, sems = rest[1:4 * NSET + 2], rest[4 * NSET + 2:]
    acc_sh = refs[4 * NSET]
    sets = tuple(
        (refs[i], refs[NSET + i], refs[2 * NSET + i], refs[3 * NSET + i],
         sems[i], sems[NSET + i], sems[2 * NSET + i])
        for i in range(NSET))
    c = lax.axis_index("c")
    s = lax.axis_index("s")
    wid = s * NC + c
    ebase = wid * EPT
    pltpu.sync_copy(w_in.at[pl.ds(ebase, EPT)], w_v)
    _zero_acc(zeros, acc_sh, s)
    plsc.subcore_barrier()
    _scatter_pipeline(h, acc_sh, packed, ebase, w_v, sets)
    plsc.subcore_barrier()
    _drain_acc(acc_sh, acc_out, c, s)


@functools.cache
def _build():
    f32 = jnp.float32
    mesh = plsc.VectorSubcoreMesh(core_axis_name="c", subcore_axis_name="s",
                                  num_cores=NC, num_subcores=NS)

    tc1a = pl.pallas_call(
        _tc1a_body,
        grid=(N // ROW_BLK,),
        in_specs=[_row_spec(), _full_spec(), _full_spec()],
        out_specs=_row_spec(),
        out_shape=jax.ShapeDtypeStruct((N, F), f32),
    )

    tc1b = pl.pallas_call(
        _tc1b_body,
        grid=(N // ROW_BLK,),
        in_specs=[_row_spec(), _full_spec()],
        out_specs=_row_spec(),
        out_shape=jax.ShapeDtypeStruct((N, F), f32),
    )

    sc_params = pltpu.CompilerParams(needs_layout_passes=False)

    scw = pl.kernel(
        _scw_body,
        out_type=jax.ShapeDtypeStruct((E,), f32),
        mesh=mesh,
        compiler_params=sc_params,
        scratch_types=(
            [pltpu.VMEM((EPT,), jnp.int32),
             pltpu.VMEM((EPT,), f32),
             pltpu.VMEM((N,), f32),
             pltpu.VMEM((N,), f32),
             pltpu.VMEM((EPT,), f32)]
            + [pltpu.SemaphoreType.DMA] * 4
        ),
    )

    tc2 = pl.pallas_call(
        _tc2_body,
        grid=(N // ROW_BLK,),
        in_specs=[_row_spec(), _row_spec(), _full_spec()],
        out_specs=_row_spec(),
        out_shape=jax.ShapeDtypeStruct((N, F), f32),
    )

    scat = pl.kernel(
        _scat_body,
        out_type=jax.ShapeDtypeStruct((NC, N, F), f32),
        mesh=mesh,
        compiler_params=sc_params,
        scratch_types=(
            [pltpu.VMEM((EPT,), f32)]
            + [pltpu.VMEM((CH, F), f32)] * NSET
            + [pltpu.VMEM((CH,), jnp.int32)] * NSET
            + [pltpu.VMEM((CH,), jnp.int32)] * NSET
            + [pltpu.VMEM((CH,), jnp.int32)] * NSET
            + [pltpu.VMEM_SHARED((N, F), f32)]
            + [pltpu.SemaphoreType.DMA] * (3 * NSET)
        ),
    )

    tc3 = pl.pallas_call(
        _tc3_body,
        grid=(N // ROW_BLK,),
        in_specs=[_row_spec(), _row_spec(), _row_spec(), _vec_spec(), _vec_spec()],
        out_specs=_row_spec(),
        out_shape=jax.ShapeDtypeStruct((N, F), f32),
    )

    return tc1a, tc1b, scw, scat, tc2, tc3


def kernel(x, edge_index, adj_vals, W1, a1, W2, ln_w, ln_b):
    tc1a, tc1b, scw, scat, tc2, tc3 = _build()
    f32 = jnp.float32
    row = edge_index[0]
    col = edge_index[1]
    packed = jnp.bitwise_or(row, jnp.left_shift(col, 16))
    a_flat = a1[:, 0]
    a_pad = jnp.zeros((F, F), f32).at[:, 0].set(a_flat[:F]).at[:, 1].set(a_flat[F:])
    zeros = jnp.zeros((N, F), f32)

    s2 = tc1a(x, W1, a_pad)
    st = s2[:, 0]
    sb = s2[:, 1]
    w = scw(packed, adj_vals, st, sb)
    h1 = tc1b(x, W1)
    acc = scat(h1, packed, w, zeros)
    h2 = tc2(acc[0], acc[1], W2)
    acc2 = scat(h2, packed, w, zeros)
    out = tc3(acc2[0], acc2[1], x, ln_w.reshape(1, F), ln_b.reshape(1, F))
    return out


# speedup vs baseline: 1.2943x; 1.2784x over previous
import functools

import jax
import jax.numpy as jnp
from jax import lax
from jax.experimental import pallas as pl
from jax.experimental.pallas import tpu as pltpu
from jax.experimental.pallas import tpu_sc as plsc

N = 10000
E = 320000
F = 128

NC = 2
NS = 16
NW = NC * NS
EPT = E // NW
CH = 80
NCHUNK = EPT // CH
RPT = 624
TAIL = N - NS * RPT

ROW_BLK = 1000


def _tc1a_body(x_ref, w1_ref, apad_ref, s2_ref):
    v = jnp.dot(w1_ref[...], apad_ref[...], preferred_element_type=jnp.float32)
    s2_ref[...] = jnp.dot(x_ref[...], v, preferred_element_type=jnp.float32)


def _tc1b_body(x_ref, w1_ref, h1_ref):
    h1_ref[...] = jnp.dot(x_ref[...], w1_ref[...],
                          preferred_element_type=jnp.float32)


def _tc2_body(a0_ref, a1_ref, w2_ref, h2_ref):
    h = jnp.maximum(a0_ref[...] + a1_ref[...], 0.0)
    h2_ref[...] = jnp.dot(h, w2_ref[...], preferred_element_type=jnp.float32)


def _tc3_body(a0_ref, a1_ref, x_ref, lnw_ref, lnb_ref, o_ref):
    h = jnp.maximum(a0_ref[...] + a1_ref[...], 0.0) + x_ref[...]
    m = jnp.mean(h, axis=-1, keepdims=True)
    cen = h - m
    var = jnp.mean(cen * cen, axis=-1, keepdims=True)
    o_ref[...] = cen * lax.rsqrt(var + 1e-5) * lnw_ref[...] + lnb_ref[...]


def _row_spec():
    return pl.BlockSpec((ROW_BLK, F), lambda i: (i, 0))


def _full_spec():
    return pl.BlockSpec((F, F), lambda i: (0, 0))


def _vec_spec():
    return pl.BlockSpec((1, F), lambda i: (0, 0))


NSET = 3


def _scatter_pipeline(h_hbm, acc_sh, packed_hbm, ebase, w_v, sets):
    def issue_p(kk, S):
        pltpu.async_copy(packed_hbm.at[pl.ds(ebase + kk * CH, CH)], S[1], S[4])

    def wait_p(S):
        pltpu.make_async_copy(packed_hbm.at[pl.ds(ebase, CH)], S[1], S[4]).wait()

    def unpack(S):
        for i in range(CH // 16):
            sl = pl.ds(i * 16, 16)
            p = S[1][sl]
            S[2][sl] = jnp.bitwise_and(p, 0xFFFF)
            S[3][sl] = lax.shift_right_logical(p, 16)

    def issue_g(S):
        pltpu.async_copy(h_hbm.at[S[3]], S[0], S[5])

    def wait_g(S):
        pltpu.make_async_copy(h_hbm.at[S[3]], S[0], S[5]).wait()

    def issue_a(S):
        pltpu.async_copy(S[0], acc_sh.at[S[2]], S[6], add=True)

    def wait_a(S):
        pltpu.make_async_copy(S[0], acc_sh.at[S[2]], S[6]).wait()

    def scale(kk, S):
        rows = S[0]

        def grp(i, _):
            wv = w_v[pl.ds(kk * CH + i * 16, 16)]
            for l in range(16):
                wb = jnp.full((16,), wv[l], dtype=jnp.float32)
                e = i * 16 + l
                for j in range(F // 16):
                    sl = pl.ds(j * 16, 16)
                    rows[e, sl] = rows[e, sl] * wb
            return 0

        lax.fori_loop(0, CH // 16, grp, 0)

    def step(k, s0, s1, s2, wait_prev_a):
        @pl.when(k + 1 < NCHUNK)
        def _():
            if wait_prev_a:
                wait_a(s1)
            wait_p(s1)
            unpack(s1)
            issue_g(s1)

        @pl.when(k + 2 < NCHUNK)
        def _():
            issue_p(k + 2, s2)

        wait_g(s0)
        scale(k, s0)
        issue_a(s0)

    def rot(k):
        return (sets[k % NSET], sets[(k + 1) % NSET], sets[(k + 2) % NSET])

    issue_p(0, sets[0])
    issue_p(1, sets[1])
    wait_p(sets[0])
    unpack(sets[0])
    issue_g(sets[0])
    for k in range(NSET - 1):
        step(k, *rot(k), False)

    def quad(t, _):
        k = NSET - 1 + NSET * t
        for d in range(NSET):
            step(k + d, *rot(NSET - 1 + d), True)
        return 0

    assert (NCHUNK - (NSET - 1)) % NSET == 0
    lax.fori_loop(0, (NCHUNK - (NSET - 1)) // NSET, quad, 0)
    for k in range(NCHUNK - NSET, NCHUNK):
        wait_a(sets[k % NSET])


def _zero_acc(zeros, acc_sh, s):
    pltpu.sync_copy(zeros.at[pl.ds(s * RPT, RPT)], acc_sh.at[pl.ds(s * RPT, RPT)])

    @pl.when(s == NS - 1)
    def _():
        pltpu.sync_copy(zeros.at[pl.ds(NS * RPT, TAIL)],
                        acc_sh.at[pl.ds(NS * RPT, TAIL)])


def _drain_acc(acc_sh, acc_out, c, s):
    pltpu.sync_copy(acc_sh.at[pl.ds(s * RPT, RPT)],
                    acc_out.at[c, pl.ds(s * RPT, RPT)])

    @pl.when(s == NS - 1)
    def _():
        pltpu.sync_copy(acc_sh.at[pl.ds(NS * RPT, TAIL)],
                        acc_out.at[c, pl.ds(NS * RPT, TAIL)])


def _scw_body(packed, adj, st, sb, w_out,
              packed_v, adj_v, st_v, sb_v, w_v,
              sem0, sem1, sem2, sem3):
    c = lax.axis_index("c")
    s = lax.axis_index("s")
    wid = s * NC + c
    ebase = wid * EPT
    d0 = pltpu.async_copy(st, st_v, sem0)
    d1 = pltpu.async_copy(sb, sb_v, sem1)
    d2 = pltpu.async_copy(packed.at[pl.ds(ebase, EPT)], packed_v, sem2)
    d3 = pltpu.async_copy(adj.at[pl.ds(ebase, EPT)], adj_v, sem3)
    d0.wait()
    d1.wait()
    d2.wait()
    d3.wait()

    def wbody(i, _):
        sl = pl.ds(i * 16, 16)
        p = packed_v[sl]
        sT = plsc.load_gather(st_v, [jnp.bitwise_and(p, 0xFFFF)])
        sB = plsc.load_gather(sb_v, [lax.shift_right_logical(p, 16)])
        sc = sT + sB
        sc = jnp.maximum(sc, 0.2 * sc)
        w_v[sl] = adj_v[sl] / (1.0 + jnp.exp(-sc))
        return 0

    lax.fori_loop(0, EPT // 16, wbody, 0)
    pltpu.sync_copy(w_v, w_out.at[pl.ds(ebase, EPT)])


def _scat_body(h, packed, w_in, zeros, acc_out, *rest):
    w_v = rest[0]
    refs, sems = rest[1:4 * NSET + 2], rest[4 * NSET + 2:]
    acc_sh = refs[4 * NSET]
    sets = tuple(
        (refs[i], refs[NSET + i], refs[2 * NSET + i], refs[3 * NSET + i],
         sems[i], sems[NSET + i], sems[2 * NSET + i])
        for i in range(NSET))
    c = lax.axis_index("c")
    s = lax.axis_index("s")
    wid = s * NC + c
    ebase = wid * EPT
    pltpu.sync_copy(w_in.at[pl.ds(ebase, EPT)], w_v)
    _zero_acc(zeros, acc_sh, s)
    plsc.subcore_barrier()
    _scatter_pipeline(h, acc_sh, packed, ebase, w_v, sets)
    plsc.subcore_barrier()
    _drain_acc(acc_sh, acc_out, c, s)


@functools.cache
def _build():
    f32 = jnp.float32
    mesh = plsc.VectorSubcoreMesh(core_axis_name="c", subcore_axis_name="s",
                                  num_cores=NC, num_subcores=NS)

    tc1a = pl.pallas_call(
        _tc1a_body,
        grid=(N // ROW_BLK,),
        in_specs=[_row_spec(), _full_spec(), _full_spec()],
        out_specs=_row_spec(),
        out_shape=jax.ShapeDtypeStruct((N, F), f32),
    )

    tc1b = pl.pallas_call(
        _tc1b_body,
        grid=(N // ROW_BLK,),
        in_specs=[_row_spec(), _full_spec()],
        out_specs=_row_spec(),
        out_shape=jax.ShapeDtypeStruct((N, F), f32),
    )

    sc_params = pltpu.CompilerParams(needs_layout_passes=False)

    scw = pl.kernel(
        _scw_body,
        out_type=jax.ShapeDtypeStruct((E,), f32),
        mesh=mesh,
        compiler_params=sc_params,
        scratch_types=(
            [pltpu.VMEM((EPT,), jnp.int32),
             pltpu.VMEM((EPT,), f32),
             pltpu.VMEM((N,), f32),
             pltpu.VMEM((N,), f32),
             pltpu.VMEM((EPT,), f32)]
            + [pltpu.SemaphoreType.DMA] * 4
        ),
    )

    tc2 = pl.pallas_call(
        _tc2_body,
        grid=(N // ROW_BLK,),
        in_specs=[_row_spec(), _row_spec(), _full_spec()],
        out_specs=_row_spec(),
        out_shape=jax.ShapeDtypeStruct((N, F), f32),
    )

    scat = pl.kernel(
        _scat_body,
        out_type=jax.ShapeDtypeStruct((NC, N, F), f32),
        mesh=mesh,
        compiler_params=sc_params,
        scratch_types=(
            [pltpu.VMEM((EPT,), f32)]
            + [pltpu.VMEM((CH, F), f32)] * NSET
            + [pltpu.VMEM((CH,), jnp.int32)] * NSET
            + [pltpu.VMEM((CH,), jnp.int32)] * NSET
            + [pltpu.VMEM((CH,), jnp.int32)] * NSET
            + [pltpu.VMEM_SHARED((N, F), f32)]
            + [pltpu.SemaphoreType.DMA] * (3 * NSET)
        ),
    )

    tc3 = pl.pallas_call(
        _tc3_body,
        grid=(N // ROW_BLK,),
        in_specs=[_row_spec(), _row_spec(), _row_spec(), _vec_spec(), _vec_spec()],
        out_specs=_row_spec(),
        out_shape=jax.ShapeDtypeStruct((N, F), f32),
    )

    return tc1a, tc1b, scw, scat, tc2, tc3


def kernel(x, edge_index, adj_vals, W1, a1, W2, ln_w, ln_b):
    tc1a, tc1b, scw, scat, tc2, tc3 = _build()
    f32 = jnp.float32
    row = edge_index[0]
    col = edge_index[1]
    packed = jnp.bitwise_or(row, jnp.left_shift(col, 16))
    a_flat = a1[:, 0]
    a_pad = jnp.zeros((F, F), f32).at[:, 0].set(a_flat[:F]).at[:, 1].set(a_flat[F:])
    zeros = jnp.zeros((N, F), f32)

    s2 = tc1a(x, W1, a_pad)
    st = s2[:, 0]
    sb = s2[:, 1]
    w = scw(packed, adj_vals, st, sb)
    h1 = tc1b(x, W1)
    acc = scat(h1, packed, w, zeros)
    h2 = tc2(acc[0], acc[1], W2)
    acc2 = scat(h2, packed, w, zeros)
    out = tc3(acc2[0], acc2[1], x, ln_w.reshape(1, F), ln_b.reshape(1, F))
    return out
